# Initial kernel scaffold; baseline (speedup 1.0000x reference)
import jax
import jax.numpy as jnp
from jax.experimental import pallas as pl


def kernel(node_feat, diff_pos, edge_index, radial, W1, b1, W2, b2):
    raise NotImplementedError("write your pallas kernel here")


# speedup vs baseline: 1.1437x; 1.1437x over previous
import functools

import jax
import jax.numpy as jnp
import numpy as np
from jax import lax
from jax.experimental import pallas as pl
from jax.experimental.pallas import tpu as pltpu

N_NODES = 10000
N_EDGES = 320000
HID = 128
RAD = 16

BN = 1000
BE = 2560

_INTERPRET = False

_S3 = float(np.sqrt(3.0))


def _silu(x):
    return x / (1.0 + jnp.exp(-x))


def _pq_body(nf_ref, w1a_ref, w1b_ref, p_ref, q_ref):
    x = nf_ref[...]
    p_ref[...] = jnp.dot(x, w1a_ref[...], preferred_element_type=jnp.float32)
    q_ref[...] = jnp.dot(x, w1b_ref[...], preferred_element_type=jnp.float32)


def _node_proj(node_feat, w1a, w1b):
    grid = N_NODES // BN
    return pl.pallas_call(
        _pq_body,
        grid=(grid,),
        in_specs=[
            pl.BlockSpec((BN, HID), lambda i: (i, 0)),
            pl.BlockSpec((HID, HID), lambda i: (0, 0)),
            pl.BlockSpec((HID, HID), lambda i: (0, 0)),
        ],
        out_specs=[
            pl.BlockSpec((BN, HID), lambda i: (i, 0)),
            pl.BlockSpec((BN, HID), lambda i: (i, 0)),
        ],
        out_shape=[
            jax.ShapeDtypeStruct((N_NODES, HID), jnp.float32),
            jax.ShapeDtypeStruct((N_NODES, HID), jnp.float32),
        ],
        interpret=_INTERPRET,
    )(node_feat, w1a, w1b)


def _edge_body(g_ref, rad_ref, dpt_ref, w1c_ref, b1_ref, w2p_ref, b2p_ref, out_ref):
    g = g_ref[...]
    rad = rad_ref[...]
    pre = g + jnp.dot(rad, w1c_ref[...], preferred_element_type=jnp.float32) + b1_ref[...]
    h = _silu(pre)
    wt = _silu(
        lax.dot_general(w2p_ref[...], h, (((0,), (1,)), ((), ())),
                        preferred_element_type=jnp.float32) + b2p_ref[...])
    d = dpt_ref[...]
    rinv = lax.rsqrt(jnp.sum(d * d, axis=0, keepdims=True))
    v = d * rinv
    vx, vy, vz = v[0:1], v[1:2], v[2:3]
    w0, w1_, w2_ = wt[0:1], wt[1:2], wt[2:3]
    one = jnp.ones_like(vx)
    zero = jnp.zeros_like(vx)
    rows = [
        w0,
        w1_ * vx, w1_ * vy, w1_ * vz,
        w2_ * (_S3 * vx * vz),
        w2_ * (_S3 * vx * vy),
        w2_ * (vy * vy - 0.5 * (vx * vx + vz * vz)),
        w2_ * (_S3 * vy * vz),
        w2_ * (0.5 * _S3 * (vz * vz - vx * vx)),
        one,
        zero, zero, zero, zero, zero, zero,
    ]
    out_t = jnp.concatenate(rows, axis=0)
    eye = jnp.eye(16, dtype=jnp.float32)
    out_ref[...] = lax.dot_general(out_t, eye, (((0,), (0,)), ((), ())),
                                   preferred_element_type=jnp.float32)


def _edge_compute(g, radial, dpt8, w1c, b1r, w2p, b2p):
    grid = N_EDGES // BE
    return pl.pallas_call(
        _edge_body,
        grid=(grid,),
        in_specs=[
            pl.BlockSpec((BE, HID), lambda i: (i, 0)),
            pl.BlockSpec((BE, RAD), lambda i: (i, 0)),
            pl.BlockSpec((8, BE), lambda i: (0, i)),
            pl.BlockSpec((RAD, HID), lambda i: (0, 0)),
            pl.BlockSpec((1, HID), lambda i: (0, 0)),
            pl.BlockSpec((HID, 8), lambda i: (0, 0)),
            pl.BlockSpec((8, 1), lambda i: (0, 0)),
        ],
        out_specs=pl.BlockSpec((BE, 16), lambda i: (i, 0)),
        out_shape=jax.ShapeDtypeStruct((N_EDGES, 16), jnp.float32),
        interpret=_INTERPRET,
    )(g, radial, dpt8, w1c, b1r, w2p, b2p)


def _fin_body(s_ref, out_ref):
    s = s_ref[0] + s_ref[1]
    cnt = jnp.maximum(s[:, 9:10], 1.0)
    out_ref[...] = s[:, 0:9] / cnt


def _finalize(partials):
    return pl.pallas_call(
        _fin_body,
        out_shape=jax.ShapeDtypeStruct((N_NODES, 9), jnp.float32),
        interpret=_INTERPRET,
    )(partials)


def kernel(node_feat, diff_pos, edge_index, radial, W1, b1, W2, b2):
    row = edge_index[0]
    col = edge_index[1]
    w1a = W1[:HID]
    w1b = W1[HID:2 * HID]
    w1c = W1[2 * HID:]
    p, q = _node_proj(node_feat, w1a, w1b)

    g = jnp.take(p, row, axis=0) + jnp.take(q, col, axis=0)

    dpt8 = jnp.concatenate(
        [diff_pos.T, jnp.zeros((5, N_EDGES), jnp.float32)], axis=0)
    w2p = jnp.concatenate([W2, jnp.zeros((HID, 5), jnp.float32)], axis=1)
    b2p = jnp.concatenate([b2, jnp.zeros((5,), jnp.float32)]).reshape(8, 1)
    out_e = _edge_compute(g, radial, dpt8, w1c, b1.reshape(1, HID), w2p, b2p)

    sums = jax.ops.segment_sum(out_e, row, num_segments=N_NODES)
    partials = jnp.stack([sums, jnp.zeros_like(sums)], axis=0)

    return _finalize(partials)


# speedup vs baseline: 2.4290x; 2.1238x over previous
import functools

import jax
import jax.numpy as jnp
import numpy as np
from jax import lax
from jax.experimental import pallas as pl
from jax.experimental.pallas import tpu as pltpu
from jax.experimental.pallas import tpu_sc as plsc

N_NODES = 10000
N_EDGES = 320000
HID = 128
RAD = 16

BN = 1000
BE = 2560

_INTERPRET = False

_S3 = float(np.sqrt(3.0))


def _silu(x):
    return x / (1.0 + jnp.exp(-x))


def _pq_body(nf_ref, w1a_ref, w1b_ref, p_ref, q_ref):
    x = nf_ref[...]
    p_ref[...] = jnp.dot(x, w1a_ref[...], preferred_element_type=jnp.float32)
    q_ref[...] = jnp.dot(x, w1b_ref[...], preferred_element_type=jnp.float32)


def _node_proj(node_feat, w1a, w1b):
    grid = N_NODES // BN
    return pl.pallas_call(
        _pq_body,
        grid=(grid,),
        in_specs=[
            pl.BlockSpec((BN, HID), lambda i: (i, 0)),
            pl.BlockSpec((HID, HID), lambda i: (0, 0)),
            pl.BlockSpec((HID, HID), lambda i: (0, 0)),
        ],
        out_specs=[
            pl.BlockSpec((BN, HID), lambda i: (i, 0)),
            pl.BlockSpec((BN, HID), lambda i: (i, 0)),
        ],
        out_shape=[
            jax.ShapeDtypeStruct((N_NODES, HID), jnp.float32),
            jax.ShapeDtypeStruct((N_NODES, HID), jnp.float32),
        ],
        interpret=_INTERPRET,
    )(node_feat, w1a, w1b)


NC = 2
NS = 16
NW = NC * NS
EPW = N_EDGES // NW
GC = 80
NCHUNK = EPW // GC


def _gather_body(p_hbm, q_hbm, row_hbm, col_hbm, g_hbm,
                 idxr_v, idxc_v, bufp_v, bufq_v, sem):
    c = lax.axis_index("c")
    s = lax.axis_index("s")
    wid = s * NC + c
    base = wid * EPW

    def chunk(k, carry):
        off = k * GC
        pltpu.sync_copy(row_hbm.at[pl.ds(base + off, GC)], idxr_v)
        pltpu.sync_copy(col_hbm.at[pl.ds(base + off, GC)], idxc_v)
        cp = pltpu.async_copy(p_hbm.at[idxr_v], bufp_v, sem)
        cq = pltpu.async_copy(q_hbm.at[idxc_v], bufq_v, sem)
        cp.wait()
        cq.wait()

        def add1(t, carry2):
            i = t // 8
            l = (t % 8) * 16
            bufp_v[i, pl.ds(l, 16)] = (bufp_v[i, pl.ds(l, 16)]
                                       + bufq_v[i, pl.ds(l, 16)])
            return carry2

        lax.fori_loop(0, GC * 8, add1, 0, unroll=8)
        pltpu.sync_copy(bufp_v, g_hbm.at[pl.ds(base + off, GC)])
        return carry

    lax.fori_loop(0, NCHUNK, chunk, 0)


def _sc_gather_add(p, q, row, col):
    mesh = plsc.VectorSubcoreMesh(core_axis_name="c", subcore_axis_name="s")
    fn = functools.partial(
        pl.kernel,
        mesh=mesh,
        out_type=jax.ShapeDtypeStruct((N_EDGES, HID), jnp.float32),
        scratch_types=[
            pltpu.VMEM((GC,), jnp.int32),
            pltpu.VMEM((GC,), jnp.int32),
            pltpu.VMEM((GC, HID), jnp.float32),
            pltpu.VMEM((GC, HID), jnp.float32),
            pltpu.SemaphoreType.DMA,
        ],
    )(_gather_body)
    return fn(p, q, row, col)


N_ACC = 10240
NPW = N_ACC // NS
SCC = 80
SK = 5
NSUPER = NCHUNK // SK
NZP = NPW // SCC


def _scatter_body(oute_hbm, row_hbm, acc_hbm,
                  idx_v, rows2_v, zrow_v, shared, sem):
    c = lax.axis_index("c")
    s = lax.axis_index("s")
    wid = s * NC + c
    base = wid * EPW

    def z1(t, carry):
        i = t // 8
        l = (t % 8) * 16
        zrow_v[i, pl.ds(l, 16)] = jnp.zeros((16,), jnp.float32)
        return carry

    lax.fori_loop(0, SCC * 8, z1, 0)

    def zpiece(i, carry):
        pltpu.sync_copy(zrow_v, shared.at[pl.ds(s * NPW + i * SCC, SCC)])
        return carry

    lax.fori_loop(0, NZP, zpiece, 0)
    plsc.subcore_barrier()

    def one(k, carry):
        e0 = base + k * SCC
        pltpu.sync_copy(row_hbm.at[pl.ds(e0, SCC)], idx_v)
        pltpu.sync_copy(oute_hbm.at[pl.ds(e0, SCC)], rows2_v)
        pltpu.sync_copy(rows2_v, shared.at[idx_v], add=True)
        return carry

    lax.fori_loop(0, NCHUNK, one, 0)
    plsc.subcore_barrier()

    def wpiece(i, carry):
        off = s * NPW + i * SCC
        pltpu.sync_copy(shared.at[pl.ds(off, SCC)], zrow_v)
        pltpu.sync_copy(zrow_v, acc_hbm.at[c].at[pl.ds(off, SCC)])
        return carry

    lax.fori_loop(0, NZP, wpiece, 0)


def _sc_scatter(out_e, row):
    mesh = plsc.VectorSubcoreMesh(core_axis_name="c", subcore_axis_name="s")
    fn = functools.partial(
        pl.kernel,
        mesh=mesh,
        out_type=jax.ShapeDtypeStruct((NC, N_ACC, 128), jnp.float32),
        scratch_types=[
            pltpu.VMEM((SCC,), jnp.int32),
            pltpu.VMEM((SCC, 128), jnp.float32),
            pltpu.VMEM((SCC, 128), jnp.float32),
            pltpu.VMEM_SHARED((N_ACC, 128), jnp.float32),
            pltpu.SemaphoreType.DMA,
        ],
    )(_scatter_body)
    return fn(out_e, row)


def _edge_body(g_ref, rad_ref, dpt_ref, w1c_ref, b1_ref, w2p_ref, b2p_ref, out_ref):
    g = g_ref[...]
    rad = rad_ref[...]
    pre = g + jnp.dot(rad, w1c_ref[...], preferred_element_type=jnp.float32) + b1_ref[...]
    h = _silu(pre)
    wt = _silu(
        lax.dot_general(w2p_ref[...], h, (((0,), (1,)), ((), ())),
                        preferred_element_type=jnp.float32) + b2p_ref[...])
    d = dpt_ref[...]
    rinv = lax.rsqrt(jnp.sum(d * d, axis=0, keepdims=True))
    v = d * rinv
    vx, vy, vz = v[0:1], v[1:2], v[2:3]
    w0, w1_, w2_ = wt[0:1], wt[1:2], wt[2:3]
    one = jnp.ones_like(vx)
    zero = jnp.zeros_like(vx)
    rows = [
        w0,
        w1_ * vx, w1_ * vy, w1_ * vz,
        w2_ * (_S3 * vx * vz),
        w2_ * (_S3 * vx * vy),
        w2_ * (vy * vy - 0.5 * (vx * vx + vz * vz)),
        w2_ * (_S3 * vy * vz),
        w2_ * (0.5 * _S3 * (vz * vz - vx * vx)),
        one,
        zero, zero, zero, zero, zero, zero,
    ]
    out_t = jnp.concatenate(rows, axis=0)
    eye = jnp.eye(16, 128, dtype=jnp.float32)
    out_ref[...] = lax.dot_general(out_t, eye, (((0,), (0,)), ((), ())),
                                   preferred_element_type=jnp.float32)


def _edge_compute(g, radial, dpt8, w1c, b1r, w2p, b2p):
    grid = N_EDGES // BE
    return pl.pallas_call(
        _edge_body,
        grid=(grid,),
        in_specs=[
            pl.BlockSpec((BE, HID), lambda i: (i, 0)),
            pl.BlockSpec((BE, RAD), lambda i: (i, 0)),
            pl.BlockSpec((8, BE), lambda i: (0, i)),
            pl.BlockSpec((RAD, HID), lambda i: (0, 0)),
            pl.BlockSpec((1, HID), lambda i: (0, 0)),
            pl.BlockSpec((HID, 8), lambda i: (0, 0)),
            pl.BlockSpec((8, 1), lambda i: (0, 0)),
        ],
        out_specs=pl.BlockSpec((BE, 128), lambda i: (i, 0)),
        out_shape=jax.ShapeDtypeStruct((N_EDGES, 128), jnp.float32),
        interpret=_INTERPRET,
    )(g, radial, dpt8, w1c, b1r, w2p, b2p)


def _fin_body(s_ref, out_ref):
    s = s_ref[0, :N_NODES, :16] + s_ref[1, :N_NODES, :16]
    cnt = jnp.maximum(s[:, 9:10], 1.0)
    out_ref[...] = s[:, 0:9] / cnt


def _finalize(partials):
    return pl.pallas_call(
        _fin_body,
        out_shape=jax.ShapeDtypeStruct((N_NODES, 9), jnp.float32),
        interpret=_INTERPRET,
    )(partials)


def kernel(node_feat, diff_pos, edge_index, radial, W1, b1, W2, b2):
    row = edge_index[0]
    col = edge_index[1]
    w1a = W1[:HID]
    w1b = W1[HID:2 * HID]
    w1c = W1[2 * HID:]
    p, q = _node_proj(node_feat, w1a, w1b)

    g = _sc_gather_add(p, q, row, col)

    dpt8 = jnp.concatenate(
        [diff_pos.T, jnp.zeros((5, N_EDGES), jnp.float32)], axis=0)
    w2p = jnp.concatenate([W2, jnp.zeros((HID, 5), jnp.float32)], axis=1)
    b2p = jnp.concatenate([b2, jnp.zeros((5,), jnp.float32)]).reshape(8, 1)
    out_e = _edge_compute(g, radial, dpt8, w1c, b1.reshape(1, HID), w2p, b2p)

    partials = _sc_scatter(out_e, row)

    return _finalize(partials)


# speedup vs baseline: 2.9292x; 1.2059x over previous
import functools

import jax
import jax.numpy as jnp
import numpy as np
from jax import lax
from jax.experimental import pallas as pl
from jax.experimental.pallas import tpu as pltpu
from jax.experimental.pallas import tpu_sc as plsc

N_NODES = 10000
N_EDGES = 320000
HID = 128
RAD = 16

BN = 1000
BE = 2560

_INTERPRET = False

_S3 = float(np.sqrt(3.0))


def _silu(x):
    return x / (1.0 + jnp.exp(-x))


def _pq_body(nf_ref, w1a_ref, w1b_ref, p_ref, q_ref):
    x = nf_ref[...]
    p_ref[...] = jnp.dot(x, w1a_ref[...], preferred_element_type=jnp.float32)
    q_ref[...] = jnp.dot(x, w1b_ref[...], preferred_element_type=jnp.float32)


def _node_proj(node_feat, w1a, w1b):
    grid = N_NODES // BN
    return pl.pallas_call(
        _pq_body,
        grid=(grid,),
        in_specs=[
            pl.BlockSpec((BN, HID), lambda i: (i, 0)),
            pl.BlockSpec((HID, HID), lambda i: (0, 0)),
            pl.BlockSpec((HID, HID), lambda i: (0, 0)),
        ],
        out_specs=[
            pl.BlockSpec((BN, HID), lambda i: (i, 0)),
            pl.BlockSpec((BN, HID), lambda i: (i, 0)),
        ],
        out_shape=[
            jax.ShapeDtypeStruct((N_NODES, HID), jnp.float32),
            jax.ShapeDtypeStruct((N_NODES, HID), jnp.float32),
        ],
        interpret=_INTERPRET,
    )(node_feat, w1a, w1b)


NC = 2
NS = 16
NW = NC * NS
EPW = N_EDGES // NW
GC = 80
NCHUNK = EPW // GC


NBUF = 5
NGRP = NCHUNK // NBUF


def _gather_body(p_hbm, q_hbm, row_hbm, col_hbm, g_hbm, *scr):
    idxr = scr[0:NBUF]
    idxc = scr[NBUF:2 * NBUF]
    bufp = scr[2 * NBUF:3 * NBUF]
    bufq = scr[3 * NBUF:4 * NBUF]
    sem_i, sem_g, sem_w = scr[4 * NBUF:4 * NBUF + 3]
    c = lax.axis_index("c")
    s = lax.axis_index("s")
    wid = s * NC + c
    base = wid * EPW

    def fire_idx(g):
        for b in range(NBUF):
            off = base + (g * NBUF + b) * GC
            pltpu.async_copy(row_hbm.at[pl.ds(off, GC)], idxr[b], sem_i)
            pltpu.async_copy(col_hbm.at[pl.ds(off, GC)], idxc[b], sem_i)

    fire_idx(0)

    def group(g, carry):
        for b in range(NBUF):
            pltpu.make_async_copy(row_hbm.at[pl.ds(base, GC)], idxr[b], sem_i).wait()
            pltpu.make_async_copy(col_hbm.at[pl.ds(base, GC)], idxc[b], sem_i).wait()

        @pl.when(g > 0)
        def _():
            for b in range(NBUF):
                pltpu.make_async_copy(bufp[b], g_hbm.at[pl.ds(base, GC)], sem_w).wait()

        handles = []
        for b in range(NBUF):
            handles.append(pltpu.async_copy(p_hbm.at[idxr[b]], bufp[b], sem_g))
            handles.append(pltpu.async_copy(q_hbm.at[idxc[b]], bufq[b], sem_g))

        for h in handles:
            h.wait()

        @pl.when(g + 1 < NGRP)
        def _():
            fire_idx(g + 1)

        for b in range(NBUF):
            bp = bufp[b]
            bq = bufq[b]

            def add1(t, carry2, bp=bp, bq=bq):
                i = t // 8
                l = (t % 8) * 16
                bp[i, pl.ds(l, 16)] = bp[i, pl.ds(l, 16)] + bq[i, pl.ds(l, 16)]
                return carry2

            lax.fori_loop(0, GC * 8, add1, 0, unroll=8)
            off = base + (g * NBUF + b) * GC
            pltpu.async_copy(bp, g_hbm.at[pl.ds(off, GC)], sem_w)
        return carry

    lax.fori_loop(0, NGRP, group, 0)
    for b in range(NBUF):
        pltpu.make_async_copy(bufp[b], g_hbm.at[pl.ds(base, GC)], sem_w).wait()


def _sc_gather_add(p, q, row, col):
    mesh = plsc.VectorSubcoreMesh(core_axis_name="c", subcore_axis_name="s")
    fn = functools.partial(
        pl.kernel,
        mesh=mesh,
        out_type=jax.ShapeDtypeStruct((N_EDGES, HID), jnp.float32),
        scratch_types=(
            [pltpu.VMEM((GC,), jnp.int32)] * (2 * NBUF)
            + [pltpu.VMEM((GC, HID), jnp.float32)] * (2 * NBUF)
            + [pltpu.SemaphoreType.DMA] * 3
        ),
    )(_gather_body)
    return fn(p, q, row, col)


N_ACC = 10240
NPW = N_ACC // NS
SCC = 80
SK = 5
NSUPER = NCHUNK // SK
NZP = NPW // SCC


def _scatter_body(oute_hbm, row_hbm, acc_hbm,
                  idx_v, rows2_v, zrow_v, shared, sem):
    c = lax.axis_index("c")
    s = lax.axis_index("s")
    wid = s * NC + c
    base = wid * EPW

    def z1(t, carry):
        i = t // 8
        l = (t % 8) * 16
        zrow_v[i, pl.ds(l, 16)] = jnp.zeros((16,), jnp.float32)
        return carry

    lax.fori_loop(0, SCC * 8, z1, 0)

    def zpiece(i, carry):
        pltpu.sync_copy(zrow_v, shared.at[pl.ds(s * NPW + i * SCC, SCC)])
        return carry

    lax.fori_loop(0, NZP, zpiece, 0)
    plsc.subcore_barrier()

    def one(k, carry):
        e0 = base + k * SCC
        pltpu.sync_copy(row_hbm.at[pl.ds(e0, SCC)], idx_v)
        pltpu.sync_copy(oute_hbm.at[pl.ds(e0, SCC)], rows2_v)
        pltpu.sync_copy(rows2_v, shared.at[idx_v], add=True)
        return carry

    lax.fori_loop(0, NCHUNK, one, 0)
    plsc.subcore_barrier()

    def wpiece(i, carry):
        off = s * NPW + i * SCC
        pltpu.sync_copy(shared.at[pl.ds(off, SCC)], zrow_v)
        pltpu.sync_copy(zrow_v, acc_hbm.at[c].at[pl.ds(off, SCC)])
        return carry

    lax.fori_loop(0, NZP, wpiece, 0)


def _sc_scatter(out_e, row):
    mesh = plsc.VectorSubcoreMesh(core_axis_name="c", subcore_axis_name="s")
    fn = functools.partial(
        pl.kernel,
        mesh=mesh,
        out_type=jax.ShapeDtypeStruct((NC, N_ACC, 128), jnp.float32),
        scratch_types=[
            pltpu.VMEM((SCC,), jnp.int32),
            pltpu.VMEM((SCC, 128), jnp.float32),
            pltpu.VMEM((SCC, 128), jnp.float32),
            pltpu.VMEM_SHARED((N_ACC, 128), jnp.float32),
            pltpu.SemaphoreType.DMA,
        ],
    )(_scatter_body)
    return fn(out_e, row)


def _edge_body(g_ref, rad_ref, dpt_ref, w1c_ref, b1_ref, w2p_ref, b2p_ref, out_ref):
    g = g_ref[...]
    rad = rad_ref[...]
    pre = g + jnp.dot(rad, w1c_ref[...], preferred_element_type=jnp.float32) + b1_ref[...]
    h = _silu(pre)
    wt = _silu(
        lax.dot_general(w2p_ref[...], h, (((0,), (1,)), ((), ())),
                        preferred_element_type=jnp.float32) + b2p_ref[...])
    d = dpt_ref[...]
    rinv = lax.rsqrt(jnp.sum(d * d, axis=0, keepdims=True))
    v = d * rinv
    vx, vy, vz = v[0:1], v[1:2], v[2:3]
    w0, w1_, w2_ = wt[0:1], wt[1:2], wt[2:3]
    one = jnp.ones_like(vx)
    zero = jnp.zeros_like(vx)
    rows = [
        w0,
        w1_ * vx, w1_ * vy, w1_ * vz,
        w2_ * (_S3 * vx * vz),
        w2_ * (_S3 * vx * vy),
        w2_ * (vy * vy - 0.5 * (vx * vx + vz * vz)),
        w2_ * (_S3 * vy * vz),
        w2_ * (0.5 * _S3 * (vz * vz - vx * vx)),
        one,
        zero, zero, zero, zero, zero, zero,
    ]
    out_t = jnp.concatenate(rows, axis=0)
    eye = jnp.eye(16, 128, dtype=jnp.float32)
    out_ref[...] = lax.dot_general(out_t, eye, (((0,), (0,)), ((), ())),
                                   preferred_element_type=jnp.float32)


def _edge_compute(g, radial, dpt8, w1c, b1r, w2p, b2p):
    grid = N_EDGES // BE
    return pl.pallas_call(
        _edge_body,
        grid=(grid,),
        in_specs=[
            pl.BlockSpec((BE, HID), lambda i: (i, 0)),
            pl.BlockSpec((BE, RAD), lambda i: (i, 0)),
            pl.BlockSpec((8, BE), lambda i: (0, i)),
            pl.BlockSpec((RAD, HID), lambda i: (0, 0)),
            pl.BlockSpec((1, HID), lambda i: (0, 0)),
            pl.BlockSpec((HID, 8), lambda i: (0, 0)),
            pl.BlockSpec((8, 1), lambda i: (0, 0)),
        ],
        out_specs=pl.BlockSpec((BE, 128), lambda i: (i, 0)),
        out_shape=jax.ShapeDtypeStruct((N_EDGES, 128), jnp.float32),
        interpret=_INTERPRET,
    )(g, radial, dpt8, w1c, b1r, w2p, b2p)


def _fin_body(s_ref, out_ref):
    s = s_ref[0, :N_NODES, :16] + s_ref[1, :N_NODES, :16]
    cnt = jnp.maximum(s[:, 9:10], 1.0)
    out_ref[...] = s[:, 0:9] / cnt


def _finalize(partials):
    return pl.pallas_call(
        _fin_body,
        out_shape=jax.ShapeDtypeStruct((N_NODES, 9), jnp.float32),
        interpret=_INTERPRET,
    )(partials)


def kernel(node_feat, diff_pos, edge_index, radial, W1, b1, W2, b2):
    row = edge_index[0]
    col = edge_index[1]
    w1a = W1[:HID]
    w1b = W1[HID:2 * HID]
    w1c = W1[2 * HID:]
    p, q = _node_proj(node_feat, w1a, w1b)

    g = _sc_gather_add(p, q, row, col)

    dpt8 = jnp.concatenate(
        [diff_pos.T, jnp.zeros((5, N_EDGES), jnp.float32)], axis=0)
    w2p = jnp.concatenate([W2, jnp.zeros((HID, 5), jnp.float32)], axis=1)
    b2p = jnp.concatenate([b2, jnp.zeros((5,), jnp.float32)]).reshape(8, 1)
    out_e = _edge_compute(g, radial, dpt8, w1c, b1.reshape(1, HID), w2p, b2p)

    partials = _sc_scatter(out_e, row)

    return _finalize(partials)


# speedup vs baseline: 4.6240x; 1.5786x over previous
import functools

import jax
import jax.numpy as jnp
import numpy as np
from jax import lax
from jax.experimental import pallas as pl
from jax.experimental.pallas import tpu as pltpu
from jax.experimental.pallas import tpu_sc as plsc

N_NODES = 10000
N_EDGES = 320000
HID = 128
RAD = 16

BN = 1000
BE = 2560

_INTERPRET = False

_S3 = float(np.sqrt(3.0))


def _silu(x):
    return x / (1.0 + jnp.exp(-x))


def _pq_body(nf_ref, w1a_ref, w1b_ref, p_ref, q_ref):
    x = nf_ref[...]
    p_ref[...] = jnp.dot(x, w1a_ref[...], preferred_element_type=jnp.float32)
    q_ref[...] = jnp.dot(x, w1b_ref[...], preferred_element_type=jnp.float32)


def _node_proj(node_feat, w1a, w1b):
    grid = N_NODES // BN
    return pl.pallas_call(
        _pq_body,
        grid=(grid,),
        in_specs=[
            pl.BlockSpec((BN, HID), lambda i: (i, 0)),
            pl.BlockSpec((HID, HID), lambda i: (0, 0)),
            pl.BlockSpec((HID, HID), lambda i: (0, 0)),
        ],
        out_specs=[
            pl.BlockSpec((BN, HID), lambda i: (i, 0)),
            pl.BlockSpec((BN, HID), lambda i: (i, 0)),
        ],
        out_shape=[
            jax.ShapeDtypeStruct((N_NODES, HID), jnp.float32),
            jax.ShapeDtypeStruct((N_NODES, HID), jnp.float32),
        ],
        interpret=_INTERPRET,
    )(node_feat, w1a, w1b)


NC = 2
NS = 16
NW = NC * NS
EPW = N_EDGES // NW
GC = 80
NCHUNK = EPW // GC


NBUF = 5
NGRP = NCHUNK // NBUF


def _gather_body(p_hbm, q_hbm, row_hbm, col_hbm, g_hbm, *scr):
    idxr = scr[0:NBUF]
    idxc = scr[NBUF:2 * NBUF]
    bufp = scr[2 * NBUF:3 * NBUF]
    bufq = scr[3 * NBUF:4 * NBUF]
    sem_i, sem_g, sem_w = scr[4 * NBUF:4 * NBUF + 3]
    c = lax.axis_index("c")
    s = lax.axis_index("s")
    wid = s * NC + c
    base = wid * EPW

    def fire_idx(g):
        for b in range(NBUF):
            off = base + (g * NBUF + b) * GC
            pltpu.async_copy(row_hbm.at[pl.ds(off, GC)], idxr[b], sem_i)
            pltpu.async_copy(col_hbm.at[pl.ds(off, GC)], idxc[b], sem_i)

    fire_idx(0)

    def group(g, carry):
        for b in range(NBUF):
            pltpu.make_async_copy(row_hbm.at[pl.ds(base, GC)], idxr[b], sem_i).wait()
            pltpu.make_async_copy(col_hbm.at[pl.ds(base, GC)], idxc[b], sem_i).wait()

        @pl.when(g > 0)
        def _():
            for b in range(NBUF):
                pltpu.make_async_copy(bufp[b], g_hbm.at[pl.ds(base, GC)], sem_w).wait()

        handles = []
        for b in range(NBUF):
            handles.append(pltpu.async_copy(p_hbm.at[idxr[b]], bufp[b], sem_g))
            handles.append(pltpu.async_copy(q_hbm.at[idxc[b]], bufq[b], sem_g))

        for h in handles:
            h.wait()

        @pl.when(g + 1 < NGRP)
        def _():
            fire_idx(g + 1)

        for b in range(NBUF):
            bp = bufp[b]
            bq = bufq[b]

            def addrow(i, carry2, bp=bp, bq=bq):
                for l in range(8):
                    plsc.addupdate(bp.at[i, pl.ds(l * 16, 16)],
                                   bq[i, pl.ds(l * 16, 16)])
                return carry2

            lax.fori_loop(0, GC, addrow, 0, unroll=4)
            off = base + (g * NBUF + b) * GC
            pltpu.async_copy(bp, g_hbm.at[pl.ds(off, GC)], sem_w)
        return carry

    lax.fori_loop(0, NGRP, group, 0)
    for b in range(NBUF):
        pltpu.make_async_copy(bufp[b], g_hbm.at[pl.ds(base, GC)], sem_w).wait()


def _sc_gather_add(p, q, row, col):
    mesh = plsc.VectorSubcoreMesh(core_axis_name="c", subcore_axis_name="s")
    fn = functools.partial(
        pl.kernel,
        mesh=mesh,
        out_type=jax.ShapeDtypeStruct((N_EDGES, HID), jnp.float32),
        scratch_types=(
            [pltpu.VMEM((GC,), jnp.int32)] * (2 * NBUF)
            + [pltpu.VMEM((GC, HID), jnp.float32)] * (2 * NBUF)
            + [pltpu.SemaphoreType.DMA] * 3
        ),
    )(_gather_body)
    return fn(p, q, row, col)


N_ACC = 10240
NPW = N_ACC // NS
SCC = 80
SK = 5
NSUPER = NCHUNK // SK
NZP = NPW // SCC


def _scatter_body(oute_hbm, row_hbm, acc_hbm, *scr):
    idx_v = scr[0:2]
    rows_v = scr[2:4]
    zrow_v, shared, sem_l, sem_a = scr[4:8]
    c = lax.axis_index("c")
    s = lax.axis_index("s")
    wid = s * NC + c
    base = wid * EPW

    def z1(t, carry):
        i = t // 8
        l = (t % 8) * 16
        zrow_v[i, pl.ds(l, 16)] = jnp.zeros((16,), jnp.float32)
        return carry

    lax.fori_loop(0, SCC * 8, z1, 0)

    def zpiece(i, carry):
        pltpu.sync_copy(zrow_v, shared.at[pl.ds(s * NPW + i * SCC, SCC)])
        return carry

    lax.fori_loop(0, NZP, zpiece, 0)
    plsc.subcore_barrier()

    def fire_loads(k, b):
        e0 = base + k * SCC
        pltpu.async_copy(row_hbm.at[pl.ds(e0, SCC)], idx_v[b], sem_l)
        pltpu.async_copy(oute_hbm.at[pl.ds(e0, SCC)], rows_v[b], sem_l)

    def wait_loads(b):
        pltpu.make_async_copy(row_hbm.at[pl.ds(base, SCC)], idx_v[b], sem_l).wait()
        pltpu.make_async_copy(oute_hbm.at[pl.ds(base, SCC)], rows_v[b], sem_l).wait()

    def wait_add(b):
        pltpu.make_async_copy(rows_v[b], shared.at[idx_v[b]], sem_a).wait()

    fire_loads(0, 0)

    def pair(gg, carry):
        for b in range(2):
            k = gg * 2 + b

            @pl.when(k > 0)
            def _(b=b):
                wait_add(1 - b)

            fire_loads(k + 1, 1 - b)
            wait_loads(b)
            pltpu.async_copy(rows_v[b], shared.at[idx_v[b]], sem_a, add=True)
        return carry

    lax.fori_loop(0, NCHUNK // 2, pair, 0)
    wait_add(1)
    wait_loads(0)
    pltpu.async_copy(rows_v[0], shared.at[idx_v[0]], sem_a, add=True)
    wait_add(0)
    plsc.subcore_barrier()

    def wpiece(i, carry):
        off = s * NPW + i * SCC
        pltpu.sync_copy(shared.at[pl.ds(off, SCC)], zrow_v)
        pltpu.sync_copy(zrow_v, acc_hbm.at[c].at[pl.ds(off, SCC)])
        return carry

    lax.fori_loop(0, NZP, wpiece, 0)


def _sc_scatter(out_e, row):
    mesh = plsc.VectorSubcoreMesh(core_axis_name="c", subcore_axis_name="s")
    fn = functools.partial(
        pl.kernel,
        mesh=mesh,
        out_type=jax.ShapeDtypeStruct((NC, N_ACC, 128), jnp.float32),
        scratch_types=[
            pltpu.VMEM((SCC,), jnp.int32),
            pltpu.VMEM((SCC,), jnp.int32),
            pltpu.VMEM((SCC, 128), jnp.float32),
            pltpu.VMEM((SCC, 128), jnp.float32),
            pltpu.VMEM((SCC, 128), jnp.float32),
            pltpu.VMEM_SHARED((N_ACC, 128), jnp.float32),
            pltpu.SemaphoreType.DMA,
            pltpu.SemaphoreType.DMA,
        ],
    )(_scatter_body)
    return fn(out_e, row)


def _edge_body(g_ref, rad_ref, dpt_ref, w1c_ref, b1_ref, w2p_ref, b2p_ref, out_ref):
    g = g_ref[...]
    rad = rad_ref[...]
    pre = g + jnp.dot(rad, w1c_ref[...], preferred_element_type=jnp.float32) + b1_ref[...]
    h = _silu(pre)
    wt = _silu(
        lax.dot_general(w2p_ref[...], h, (((0,), (1,)), ((), ())),
                        preferred_element_type=jnp.float32) + b2p_ref[...])
    d = dpt_ref[...]
    rinv = lax.rsqrt(jnp.sum(d * d, axis=0, keepdims=True))
    v = d * rinv
    vx, vy, vz = v[0:1], v[1:2], v[2:3]
    w0, w1_, w2_ = wt[0:1], wt[1:2], wt[2:3]
    one = jnp.ones_like(vx)
    zero = jnp.zeros_like(vx)
    rows = [
        w0,
        w1_ * vx, w1_ * vy, w1_ * vz,
        w2_ * (_S3 * vx * vz),
        w2_ * (_S3 * vx * vy),
        w2_ * (vy * vy - 0.5 * (vx * vx + vz * vz)),
        w2_ * (_S3 * vy * vz),
        w2_ * (0.5 * _S3 * (vz * vz - vx * vx)),
        one,
        zero, zero, zero, zero, zero, zero,
    ]
    out_t = jnp.concatenate(rows, axis=0)
    eye = jnp.eye(16, 128, dtype=jnp.float32)
    out_ref[...] = lax.dot_general(out_t, eye, (((0,), (0,)), ((), ())),
                                   preferred_element_type=jnp.float32)


def _edge_compute(g, radial, dpt8, w1c, b1r, w2p, b2p):
    grid = N_EDGES // BE
    return pl.pallas_call(
        _edge_body,
        grid=(grid,),
        in_specs=[
            pl.BlockSpec((BE, HID), lambda i: (i, 0)),
            pl.BlockSpec((BE, RAD), lambda i: (i, 0)),
            pl.BlockSpec((8, BE), lambda i: (0, i)),
            pl.BlockSpec((RAD, HID), lambda i: (0, 0)),
            pl.BlockSpec((1, HID), lambda i: (0, 0)),
            pl.BlockSpec((HID, 8), lambda i: (0, 0)),
            pl.BlockSpec((8, 1), lambda i: (0, 0)),
        ],
        out_specs=pl.BlockSpec((BE, 128), lambda i: (i, 0)),
        out_shape=jax.ShapeDtypeStruct((N_EDGES, 128), jnp.float32),
        interpret=_INTERPRET,
    )(g, radial, dpt8, w1c, b1r, w2p, b2p)


def _fin_body(s_ref, out_ref):
    s = s_ref[0, :N_NODES, :16] + s_ref[1, :N_NODES, :16]
    cnt = jnp.maximum(s[:, 9:10], 1.0)
    out_ref[...] = s[:, 0:9] / cnt


def _finalize(partials):
    return pl.pallas_call(
        _fin_body,
        out_shape=jax.ShapeDtypeStruct((N_NODES, 9), jnp.float32),
        interpret=_INTERPRET,
    )(partials)


def kernel(node_feat, diff_pos, edge_index, radial, W1, b1, W2, b2):
    row = edge_index[0]
    col = edge_index[1]
    w1a = W1[:HID]
    w1b = W1[HID:2 * HID]
    w1c = W1[2 * HID:]
    p, q = _node_proj(node_feat, w1a, w1b)

    g = _sc_gather_add(p, q, row, col)

    dpt8 = jnp.concatenate(
        [diff_pos.T, jnp.zeros((5, N_EDGES), jnp.float32)], axis=0)
    w2p = jnp.concatenate([W2, jnp.zeros((HID, 5), jnp.float32)], axis=1)
    b2p = jnp.concatenate([b2, jnp.zeros((5,), jnp.float32)]).reshape(8, 1)
    out_e = _edge_compute(g, radial, dpt8, w1c, b1.reshape(1, HID), w2p, b2p)

    partials = _sc_scatter(out_e, row)

    return _finalize(partials)


# speedup vs baseline: 4.7887x; 1.0356x over previous
import functools

import jax
import jax.numpy as jnp
import numpy as np
from jax import lax
from jax.experimental import pallas as pl
from jax.experimental.pallas import tpu as pltpu
from jax.experimental.pallas import tpu_sc as plsc

N_NODES = 10000
N_EDGES = 320000
HID = 128
RAD = 16

BN = 1000
BE = 2560

_INTERPRET = False

_S3 = float(np.sqrt(3.0))


def _silu(x):
    return x / (1.0 + jnp.exp(-x))


def _pq_body(nf_ref, w1a_ref, w1b_ref, p_ref, q_ref):
    x = nf_ref[...]
    p_ref[...] = jnp.dot(x, w1a_ref[...], preferred_element_type=jnp.float32)
    q_ref[...] = jnp.dot(x, w1b_ref[...], preferred_element_type=jnp.float32)


def _node_proj(node_feat, w1a, w1b):
    grid = N_NODES // BN
    return pl.pallas_call(
        _pq_body,
        grid=(grid,),
        in_specs=[
            pl.BlockSpec((BN, HID), lambda i: (i, 0)),
            pl.BlockSpec((HID, HID), lambda i: (0, 0)),
            pl.BlockSpec((HID, HID), lambda i: (0, 0)),
        ],
        out_specs=[
            pl.BlockSpec((BN, HID), lambda i: (i, 0)),
            pl.BlockSpec((BN, HID), lambda i: (i, 0)),
        ],
        out_shape=[
            jax.ShapeDtypeStruct((N_NODES, HID), jnp.float32),
            jax.ShapeDtypeStruct((N_NODES, HID), jnp.float32),
        ],
        interpret=_INTERPRET,
    )(node_feat, w1a, w1b)


NC = 2
NS = 16
NW = NC * NS
EPW = N_EDGES // NW
GC = 80
NCHUNK = EPW // GC


NBUF = 5
NGRP = NCHUNK // NBUF


def _gather_body(p_hbm, q_hbm, row_hbm, col_hbm, g_hbm, *scr):
    idxr = scr[0:NBUF]
    idxc = scr[NBUF:2 * NBUF]
    bufp = scr[2 * NBUF:3 * NBUF]
    bufq = scr[3 * NBUF:4 * NBUF]
    sem_i, sem_g, sem_w = scr[4 * NBUF:4 * NBUF + 3]
    c = lax.axis_index("c")
    s = lax.axis_index("s")
    wid = s * NC + c
    base = wid * EPW

    def fire_idx(g):
        for b in range(NBUF):
            off = base + (g * NBUF + b) * GC
            pltpu.async_copy(row_hbm.at[pl.ds(off, GC)], idxr[b], sem_i)
            pltpu.async_copy(col_hbm.at[pl.ds(off, GC)], idxc[b], sem_i)

    fire_idx(0)

    def group(g, carry):
        for b in range(NBUF):
            pltpu.make_async_copy(row_hbm.at[pl.ds(base, GC)], idxr[b], sem_i).wait()
            pltpu.make_async_copy(col_hbm.at[pl.ds(base, GC)], idxc[b], sem_i).wait()

        @pl.when(g > 0)
        def _():
            for b in range(NBUF):
                pltpu.make_async_copy(bufp[b], g_hbm.at[pl.ds(base, GC)], sem_w).wait()

        handles = []
        for b in range(NBUF):
            handles.append(pltpu.async_copy(p_hbm.at[idxr[b]], bufp[b], sem_g))
            handles.append(pltpu.async_copy(q_hbm.at[idxc[b]], bufq[b], sem_g))

        for b in range(NBUF):
            handles[2 * b].wait()
            handles[2 * b + 1].wait()
            if b == NBUF - 1:
                @pl.when(g + 1 < NGRP)
                def _():
                    fire_idx(g + 1)
            bp = bufp[b]
            bq = bufq[b]

            def addrow(i, carry2, bp=bp, bq=bq):
                for l in range(8):
                    plsc.addupdate(bp.at[i, pl.ds(l * 16, 16)],
                                   bq[i, pl.ds(l * 16, 16)])
                return carry2

            lax.fori_loop(0, GC, addrow, 0, unroll=4)
            off = base + (g * NBUF + b) * GC
            pltpu.async_copy(bp, g_hbm.at[pl.ds(off, GC)], sem_w)
        return carry

    lax.fori_loop(0, NGRP, group, 0)
    for b in range(NBUF):
        pltpu.make_async_copy(bufp[b], g_hbm.at[pl.ds(base, GC)], sem_w).wait()


def _sc_gather_add(p, q, row, col):
    mesh = plsc.VectorSubcoreMesh(core_axis_name="c", subcore_axis_name="s")
    fn = functools.partial(
        pl.kernel,
        mesh=mesh,
        out_type=jax.ShapeDtypeStruct((N_EDGES, HID), jnp.float32),
        scratch_types=(
            [pltpu.VMEM((GC,), jnp.int32)] * (2 * NBUF)
            + [pltpu.VMEM((GC, HID), jnp.float32)] * (2 * NBUF)
            + [pltpu.SemaphoreType.DMA] * 3
        ),
    )(_gather_body)
    return fn(p, q, row, col)


WOUT = 128
N_ACC = 10240
NPW = N_ACC // NS
SCC = 80
SK = 5
NSUPER = NCHUNK // SK
NZP = NPW // SCC


def _scatter_body(oute_hbm, row_hbm, acc_hbm, *scr):
    idx_v = scr[0:2]
    rows_v = scr[2:4]
    zrow_v, shared, sem_l, sem_a = scr[4:8]
    c = lax.axis_index("c")
    s = lax.axis_index("s")
    wid = s * NC + c
    base = wid * EPW

    def z1(t, carry):
        i = t // 8
        l = (t % 8) * 16
        zrow_v[i, pl.ds(l, 16)] = jnp.zeros((16,), jnp.float32)
        return carry

    lax.fori_loop(0, SCC * 8, z1, 0)

    def zpiece(i, carry):
        pltpu.sync_copy(zrow_v, shared.at[pl.ds(s * NPW + i * SCC, SCC)])
        return carry

    lax.fori_loop(0, NZP, zpiece, 0)
    plsc.subcore_barrier()

    def fire_loads(k, b):
        e0 = base + k * SCC
        pltpu.async_copy(row_hbm.at[pl.ds(e0, SCC)], idx_v[b], sem_l)
        pltpu.async_copy(oute_hbm.at[pl.ds(e0, SCC)], rows_v[b], sem_l)

    def wait_loads(b):
        pltpu.make_async_copy(row_hbm.at[pl.ds(base, SCC)], idx_v[b], sem_l).wait()
        pltpu.make_async_copy(oute_hbm.at[pl.ds(base, SCC)], rows_v[b], sem_l).wait()

    def wait_add(b):
        pltpu.make_async_copy(rows_v[b], shared.at[idx_v[b]], sem_a).wait()

    fire_loads(0, 0)

    def pair(gg, carry):
        for b in range(2):
            k = gg * 2 + b

            @pl.when(k > 0)
            def _(b=b):
                wait_add(1 - b)

            fire_loads(k + 1, 1 - b)
            wait_loads(b)
            pltpu.async_copy(rows_v[b], shared.at[idx_v[b]], sem_a, add=True)
        return carry

    lax.fori_loop(0, NCHUNK // 2, pair, 0)
    wait_add(1)
    wait_loads(0)
    pltpu.async_copy(rows_v[0], shared.at[idx_v[0]], sem_a, add=True)
    wait_add(0)
    plsc.subcore_barrier()

    def wpiece(i, carry):
        off = s * NPW + i * SCC
        pltpu.sync_copy(shared.at[pl.ds(off, SCC)], zrow_v)
        pltpu.sync_copy(zrow_v, acc_hbm.at[c].at[pl.ds(off, SCC)])
        return carry

    lax.fori_loop(0, NZP, wpiece, 0)


def _sc_scatter(out_e, row):
    mesh = plsc.VectorSubcoreMesh(core_axis_name="c", subcore_axis_name="s")
    fn = functools.partial(
        pl.kernel,
        mesh=mesh,
        out_type=jax.ShapeDtypeStruct((NC, N_ACC, WOUT), jnp.float32),
        scratch_types=[
            pltpu.VMEM((SCC,), jnp.int32),
            pltpu.VMEM((SCC,), jnp.int32),
            pltpu.VMEM((SCC, WOUT), jnp.float32),
            pltpu.VMEM((SCC, WOUT), jnp.float32),
            pltpu.VMEM((SCC, WOUT), jnp.float32),
            pltpu.VMEM_SHARED((N_ACC, WOUT), jnp.float32),
            pltpu.SemaphoreType.DMA,
            pltpu.SemaphoreType.DMA,
        ],
    )(_scatter_body)
    return fn(out_e, row)


def _edge_body(g_ref, rad_ref, dpt_ref, w1c_ref, b1_ref, w2p_ref, b2p_ref, out_ref):
    g = g_ref[...]
    rad = rad_ref[...]
    pre = g + jnp.dot(rad, w1c_ref[...], preferred_element_type=jnp.float32) + b1_ref[...]
    h = _silu(pre)
    wt = _silu(
        lax.dot_general(w2p_ref[...], h, (((0,), (1,)), ((), ())),
                        preferred_element_type=jnp.float32) + b2p_ref[...])
    d = dpt_ref[...]
    rinv = lax.rsqrt(jnp.sum(d * d, axis=0, keepdims=True))
    v = d * rinv
    vx, vy, vz = v[0:1], v[1:2], v[2:3]
    w0, w1_, w2_ = wt[0:1], wt[1:2], wt[2:3]
    one = jnp.ones_like(vx)
    zero = jnp.zeros_like(vx)
    rows = [
        w0,
        w1_ * vx, w1_ * vy, w1_ * vz,
        w2_ * (_S3 * vx * vz),
        w2_ * (_S3 * vx * vy),
        w2_ * (vy * vy - 0.5 * (vx * vx + vz * vz)),
        w2_ * (_S3 * vy * vz),
        w2_ * (0.5 * _S3 * (vz * vz - vx * vx)),
        one,
        zero, zero, zero, zero, zero, zero,
    ]
    out_t = jnp.concatenate(rows, axis=0)
    eye = jnp.eye(16, WOUT, dtype=jnp.float32)
    out_ref[...] = lax.dot_general(out_t, eye, (((0,), (0,)), ((), ())),
                                   preferred_element_type=jnp.float32)


def _edge_compute(g, radial, dpt8, w1c, b1r, w2p, b2p):
    grid = N_EDGES // BE
    return pl.pallas_call(
        _edge_body,
        grid=(grid,),
        in_specs=[
            pl.BlockSpec((BE, HID), lambda i: (i, 0)),
            pl.BlockSpec((BE, RAD), lambda i: (i, 0)),
            pl.BlockSpec((8, BE), lambda i: (0, i)),
            pl.BlockSpec((RAD, HID), lambda i: (0, 0)),
            pl.BlockSpec((1, HID), lambda i: (0, 0)),
            pl.BlockSpec((HID, 8), lambda i: (0, 0)),
            pl.BlockSpec((8, 1), lambda i: (0, 0)),
        ],
        out_specs=pl.BlockSpec((BE, WOUT), lambda i: (i, 0)),
        out_shape=jax.ShapeDtypeStruct((N_EDGES, WOUT), jnp.float32),
        interpret=_INTERPRET,
    )(g, radial, dpt8, w1c, b1r, w2p, b2p)


def _fin_body(s_ref, out_ref):
    s = s_ref[0, :N_NODES, :16] + s_ref[1, :N_NODES, :16]
    cnt = jnp.maximum(s[:, 9:10], 1.0)
    out_ref[...] = s[:, 0:9] / cnt


def _finalize(partials):
    return pl.pallas_call(
        _fin_body,
        out_shape=jax.ShapeDtypeStruct((N_NODES, 9), jnp.float32),
        interpret=_INTERPRET,
    )(partials)


def kernel(node_feat, diff_pos, edge_index, radial, W1, b1, W2, b2):
    row = edge_index[0]
    col = edge_index[1]
    w1a = W1[:HID]
    w1b = W1[HID:2 * HID]
    w1c = W1[2 * HID:]
    p, q = _node_proj(node_feat, w1a, w1b)

    g = _sc_gather_add(p, q, row, col)

    dpt8 = jnp.concatenate(
        [diff_pos.T, jnp.zeros((5, N_EDGES), jnp.float32)], axis=0)
    w2p = jnp.concatenate([W2, jnp.zeros((HID, 5), jnp.float32)], axis=1)
    b2p = jnp.concatenate([b2, jnp.zeros((5,), jnp.float32)]).reshape(8, 1)
    out_e = _edge_compute(g, radial, dpt8, w1c, b1.reshape(1, HID), w2p, b2p)

    partials = _sc_scatter(out_e, row)

    return _finalize(partials)
